# baseline (speedup 1.0000x reference)
import jax
import jax.numpy as jnp
from jax.experimental import pallas as pl
from jax.experimental.pallas import tpu as pltpu

_ROWS_PER_BLOCK = 8


def _clipped_row_index(seq_len, n_rows):
    return jnp.clip(seq_len - 1, 0, n_rows - 1)


def kernel(seq_len, table):
    n_rows, d_model = table.shape
    static_len = (n_rows + 1) // 2
    sl = jnp.asarray(seq_len, jnp.int32).reshape(1)

    def body(sl_ref, tile_ref, out_ref):
        idx = _clipped_row_index(sl_ref[0], n_rows)
        row = tile_ref[pl.ds(idx % _ROWS_PER_BLOCK, 1), :]
        col = row.reshape(d_model, 1)
        out_ref[...] = jnp.broadcast_to(col, out_ref.shape)

    n_col_blocks = 4
    grid_spec = pltpu.PrefetchScalarGridSpec(
        num_scalar_prefetch=1,
        grid=(n_col_blocks,),
        in_specs=[
            pl.BlockSpec(
                (_ROWS_PER_BLOCK, d_model),
                lambda i, sl_ref: (
                    _clipped_row_index(sl_ref[0], n_rows) // _ROWS_PER_BLOCK,
                    0,
                ),
            )
        ],
        out_specs=pl.BlockSpec(
            (d_model, static_len // n_col_blocks), lambda i, sl_ref: (0, i)
        ),
    )
    return pl.pallas_call(
        body,
        grid_spec=grid_spec,
        out_shape=jax.ShapeDtypeStruct((d_model, static_len), table.dtype),
    )(sl, table)


# speedup vs baseline: 1.3435x; 1.3435x over previous
import jax
import jax.numpy as jnp
from jax.experimental import pallas as pl
from jax.experimental.pallas import tpu as pltpu

_ROWS_PER_BLOCK = 8


def _clipped_row_index(seq_len, n_rows):
    return jnp.clip(seq_len - 1, 0, n_rows - 1)


def kernel(seq_len, table):
    n_rows, d_model = table.shape
    static_len = (n_rows + 1) // 2
    sl = jnp.asarray(seq_len, jnp.int32).reshape(1)

    def body(sl_ref, tile_ref, out_ref):
        idx = _clipped_row_index(sl_ref[0], n_rows)
        row = tile_ref[pl.ds(idx % _ROWS_PER_BLOCK, 1), :]
        col = row.reshape(d_model, 1)
        out_ref[...] = jnp.broadcast_to(col, out_ref.shape)

    grid_spec = pltpu.PrefetchScalarGridSpec(
        num_scalar_prefetch=1,
        grid=(1,),
        in_specs=[
            pl.BlockSpec(
                (_ROWS_PER_BLOCK, d_model),
                lambda i, sl_ref: (
                    _clipped_row_index(sl_ref[0], n_rows) // _ROWS_PER_BLOCK,
                    0,
                ),
            )
        ],
        out_specs=pl.BlockSpec(
            (d_model, static_len), lambda i, sl_ref: (0, 0)
        ),
    )
    return pl.pallas_call(
        body,
        grid_spec=grid_spec,
        out_shape=jax.ShapeDtypeStruct((d_model, static_len), table.dtype),
    )(sl, table)
